# Initial kernel scaffold; baseline (speedup 1.0000x reference)
import jax
import jax.numpy as jnp
from jax.experimental import pallas as pl


def kernel(x, user_embedding, SW1, Sb1, SW2, Sb2, EW1, Eb1, EW2, Eb2, UW1, Ub1, UW2, Ub2):
    raise NotImplementedError("write your pallas kernel here")


# speedup vs baseline: 8.9336x; 8.9336x over previous
import jax
import jax.numpy as jnp
from jax.experimental import pallas as pl

B, D, E = 4096, 64, 8
S1, S2 = 32, 8


def _moe_ffn_kernel(x_ref, ue_ref, sw1t_ref, sb1_ref, sw2t_ref, sb2_ref,
                    ew1t_ref, eb1_ref, ew2t_ref, eb2_ref,
                    uw1t_ref, ub1_ref, uw2t_ref, ub2_ref, o_ref):
    f32 = jnp.float32
    x = x_ref[...]

    h = jnp.maximum(
        jnp.dot(ue_ref[...], sw1t_ref[...], preferred_element_type=f32)
        + sb1_ref[...], 0.0)
    logits = (jnp.dot(h, sw2t_ref[...], preferred_element_type=f32)
              + sb2_ref[...])
    routes = jnp.argmax(logits, axis=1).reshape(-1, 1)

    uh = jnp.maximum(
        jnp.dot(x, uw1t_ref[...], preferred_element_type=f32)
        + ub1_ref[...], 0.0)
    acc = (jnp.dot(uh, uw2t_ref[...], preferred_element_type=f32)
           + ub2_ref[...])

    for e in range(E):
        h1 = jnp.maximum(
            jnp.dot(x, ew1t_ref[e], preferred_element_type=f32)
            + eb1_ref[e][None, :], 0.0)
        oe = (jnp.dot(h1, ew2t_ref[e], preferred_element_type=f32)
              + eb2_ref[e][None, :])
        acc = acc + jnp.where(routes == e, oe, 0.0)

    o_ref[...] = acc


def kernel(x, user_embedding, SW1, Sb1, SW2, Sb2, EW1, Eb1, EW2, Eb2,
           UW1, Ub1, UW2, Ub2):
    out = pl.pallas_call(
        _moe_ffn_kernel,
        out_shape=jax.ShapeDtypeStruct((B, D), jnp.float32),
    )(x, user_embedding,
      SW1.T, Sb1.reshape(1, S1), SW2.T, Sb2.reshape(1, S2),
      EW1.transpose(0, 2, 1), Eb1, EW2.transpose(0, 2, 1), Eb2,
      UW1.T, Ub1.reshape(1, D), UW2.T, Ub2.reshape(1, D))
    return out


# speedup vs baseline: 9.6620x; 1.0815x over previous
import jax
import jax.numpy as jnp
from jax.experimental import pallas as pl

B, D, E = 4096, 64, 8
S1, S2 = 32, 8
NBLK = E + 1
H = NBLK * D


def _moe_ffn_kernel(x_ref, ue_ref, sw1t_ref, sb1_ref, sw2t_ref, sb2_ref,
                    w1cat_ref, b1cat_ref, w2stack_ref, eb2_ref, ub2_ref,
                    o_ref):
    f32 = jnp.float32
    x = x_ref[...]

    h = jnp.maximum(
        jnp.dot(ue_ref[...], sw1t_ref[...], preferred_element_type=f32)
        + sb1_ref[...], 0.0)
    logits = (jnp.dot(h, sw2t_ref[...], preferred_element_type=f32)
              + sb2_ref[...])
    routes = jnp.argmax(logits, axis=1).reshape(-1, 1)

    h1 = jnp.maximum(
        jnp.dot(x, w1cat_ref[...], preferred_element_type=f32)
        + b1cat_ref[...], 0.0)

    blk = jax.lax.broadcasted_iota(jnp.int32, (B, H), 1) >> 6
    h1 = jnp.where((blk == routes) | (blk == E), h1, 0.0)

    out = jnp.dot(h1, w2stack_ref[...], preferred_element_type=f32)

    onehot = (jax.lax.broadcasted_iota(jnp.int32, (B, E), 1)
              == routes).astype(f32)
    out = out + jnp.dot(onehot, eb2_ref[...],
                        preferred_element_type=f32) + ub2_ref[...]
    o_ref[...] = out


def kernel(x, user_embedding, SW1, Sb1, SW2, Sb2, EW1, Eb1, EW2, Eb2,
           UW1, Ub1, UW2, Ub2):
    w1cat = jnp.concatenate(
        [EW1.transpose(0, 2, 1).transpose(1, 0, 2).reshape(D, E * D),
         UW1.T], axis=1)
    b1cat = jnp.concatenate([Eb1.reshape(1, E * D), Ub1.reshape(1, D)],
                            axis=1)
    w2stack = jnp.concatenate([EW2.transpose(0, 2, 1).reshape(E * D, D),
                               UW2.T], axis=0)
    out = pl.pallas_call(
        _moe_ffn_kernel,
        out_shape=jax.ShapeDtypeStruct((B, D), jnp.float32),
    )(x, user_embedding,
      SW1.T, Sb1.reshape(1, S1), SW2.T, Sb2.reshape(1, S2),
      w1cat, b1cat, w2stack, Eb2, Ub2.reshape(1, D))
    return out


# speedup vs baseline: 13.6343x; 1.4111x over previous
import jax
import jax.numpy as jnp
from jax.experimental import pallas as pl

B, D, E = 4096, 64, 8
S1, S2 = 32, 8

_DN = (((1,), (1,)), ((), ()))
_F32 = jnp.float32


def _dot_t(a, w):
    return jax.lax.dot_general(a, w, _DN, preferred_element_type=_F32)


def _moe_ffn_kernel(x_ref, ue_ref, sw1_ref, sb1_ref, sw2_ref, sb2_ref,
                    ew1_ref, eb1_ref, ew2_ref, eb2_ref,
                    uw1_ref, ub1_ref, uw2_ref, ub2_ref, o_ref):
    x = x_ref[...]

    h = jnp.maximum(_dot_t(ue_ref[...], sw1_ref[...]) + sb1_ref[...], 0.0)
    logits = _dot_t(h, sw2_ref[...]) + sb2_ref[...]
    routes = jnp.argmax(logits, axis=1).reshape(-1, 1)

    uh = jnp.maximum(_dot_t(x, uw1_ref[...]) + ub1_ref[...], 0.0)
    acc = _dot_t(uh, uw2_ref[...]) + ub2_ref[...]

    for e in range(E):
        h1 = jnp.maximum(_dot_t(x, ew1_ref[e]) + eb1_ref[e][None, :], 0.0)
        oe = _dot_t(h1, ew2_ref[e]) + eb2_ref[e][None, :]
        acc = acc + jnp.where(routes == e, oe, 0.0)

    o_ref[...] = acc


def kernel(x, user_embedding, SW1, Sb1, SW2, Sb2, EW1, Eb1, EW2, Eb2,
           UW1, Ub1, UW2, Ub2):
    out = pl.pallas_call(
        _moe_ffn_kernel,
        out_shape=jax.ShapeDtypeStruct((B, D), jnp.float32),
    )(x, user_embedding,
      SW1, Sb1.reshape(1, S1), SW2, Sb2.reshape(1, S2),
      EW1, Eb1, EW2, Eb2,
      UW1, Ub1.reshape(1, D), UW2, Ub2.reshape(1, D))
    return out


# speedup vs baseline: 15.1773x; 1.1132x over previous
import jax
import jax.numpy as jnp
from jax.experimental import pallas as pl

B, D, E = 4096, 64, 8
S1, S2 = 32, 8
NBLK = E + 1
H = NBLK * D

_DN = (((1,), (1,)), ((), ()))
_F32 = jnp.float32


def _dot_t(a, w):
    return jax.lax.dot_general(a, w, _DN, preferred_element_type=_F32)


def _moe_ffn_kernel(x_ref, ue_ref, sw1_ref, sb1_ref, sw2_ref, sb2_ref,
                    ew1_ref, eb1_ref, ew2_ref, eb2_ref,
                    uw1_ref, ub1_ref, uw2_ref, ub2_ref, o_ref):
    x = x_ref[...]

    h = jnp.maximum(_dot_t(ue_ref[...], sw1_ref[...])
                    + sb1_ref[...][None, :], 0.0)
    logits = _dot_t(h, sw2_ref[...]) + sb2_ref[...][None, :]
    routes = jnp.argmax(logits, axis=1).reshape(-1, 1)

    w1cat = jnp.concatenate(
        [ew1_ref[e].T for e in range(E)] + [uw1_ref[...].T], axis=1)
    b1cat = jnp.concatenate(
        [eb1_ref[e][None, :] for e in range(E)] + [ub1_ref[...][None, :]],
        axis=1)
    w2stack = jnp.concatenate(
        [ew2_ref[e].T for e in range(E)] + [uw2_ref[...].T], axis=0)

    h1 = jnp.maximum(
        jnp.dot(x, w1cat, preferred_element_type=_F32) + b1cat, 0.0)

    blk = jax.lax.broadcasted_iota(jnp.int32, (B, H), 1) >> 6
    h1 = jnp.where((blk == routes) | (blk == E), h1, 0.0)

    out = jnp.dot(h1, w2stack, preferred_element_type=_F32)

    onehot = (jax.lax.broadcasted_iota(jnp.int32, (B, E), 1)
              == routes).astype(_F32)
    out = out + (jnp.dot(onehot, eb2_ref[...], preferred_element_type=_F32)
                 + ub2_ref[...][None, :])
    o_ref[...] = out


def kernel(x, user_embedding, SW1, Sb1, SW2, Sb2, EW1, Eb1, EW2, Eb2,
           UW1, Ub1, UW2, Ub2):
    out = pl.pallas_call(
        _moe_ffn_kernel,
        out_shape=jax.ShapeDtypeStruct((B, D), jnp.float32),
    )(x, user_embedding, SW1, Sb1, SW2, Sb2, EW1, Eb1, EW2, Eb2,
      UW1, Ub1, UW2, Ub2)
    return out
